# Initial kernel scaffold; baseline (speedup 1.0000x reference)
import jax
import jax.numpy as jnp
from jax.experimental import pallas as pl


def kernel(x, edge_index, W1, a_src1, a_dst1, W2, a_src2, a_dst2):
    raise NotImplementedError("write your pallas kernel here")


# speedup vs baseline: 44.9861x; 44.9861x over previous
import functools

import jax
import jax.numpy as jnp
from jax import lax
from jax.experimental import pallas as pl
from jax.experimental.pallas import tpu as pltpu
from jax.experimental.pallas import tpu_sc as plsc

N = 10000
E = 320000
DIM_IN = 128
DIM_H = 8
HEADS = 8
C1 = HEADS * DIM_H
DIM_OUT = 16

NC = 2
NS = 16
NW = NC * NS
CHUNK = 128
NCHUNK = E // CHUNK
BASE_CH = NCHUNK // NW
REM_CH = NCHUNK % NW
RPT = 624
TAIL0 = NS * RPT
TAILR = N - TAIL0


def _edge_kernel(cw, dh):
  tw = cw + 16
  ngrp = cw // 16
  hpg = 16 // dh

  mesh = plsc.VectorSubcoreMesh(core_axis_name="c", subcore_axis_name="s")

  @functools.partial(
      pl.kernel,
      out_type=(
          jax.ShapeDtypeStruct((NC, N, cw), jnp.float32),
          jax.ShapeDtypeStruct((NC, N, 16), jnp.float32),
      ),
      mesh=mesh,
      scratch_types=[
          pltpu.VMEM((CHUNK,), jnp.int32),
          pltpu.VMEM((CHUNK,), jnp.int32),
          pltpu.VMEM((CHUNK, tw), jnp.float32),
          pltpu.VMEM((CHUNK, 16), jnp.float32),
          pltpu.VMEM((CHUNK, cw), jnp.float32),
          pltpu.VMEM((CHUNK, 16), jnp.float32),
          pltpu.VMEM_SHARED((N, cw), jnp.float32),
          pltpu.VMEM_SHARED((N, 16), jnp.float32),
          pltpu.SemaphoreType.DMA,
      ],
      compiler_params=pltpu.CompilerParams(
          use_tc_tiling_on_sc=False, needs_layout_passes=False
      ),
  )
  def k(t_hbm, adt_hbm, ei_hbm, zh_hbm, zp_hbm, ph_hbm, pp_hbm,
        sidx, didx, g, gd, msg, pbuf, acc_h, acc_p, sem):
    c = lax.axis_index("c")
    s = lax.axis_index("s")
    w = s * NC + c
    r0 = s * RPT

    pltpu.sync_copy(zh_hbm, acc_h.at[pl.ds(r0, RPT)])
    pltpu.sync_copy(zp_hbm, acc_p.at[pl.ds(r0, RPT)])

    @pl.when(s == NS - 1)
    def _():
      pltpu.sync_copy(zh_hbm.at[pl.ds(0, TAILR)], acc_h.at[pl.ds(TAIL0, TAILR)])
      pltpu.sync_copy(zp_hbm.at[pl.ds(0, TAILR)], acc_p.at[pl.ds(TAIL0, TAILR)])

    plsc.subcore_barrier()

    lane = lax.iota(jnp.int32, 16)
    shift = dh.bit_length() - 1
    pidx = [(lane >> shift) + g_ * hpg for g_ in range(ngrp)]

    nch = BASE_CH + jnp.where(w < REM_CH, 1, 0)

    def chunk_body(i, carry):
      off = (w + i * NW) * CHUNK
      pltpu.sync_copy(ei_hbm.at[pl.ds(off, CHUNK)], sidx)
      pltpu.sync_copy(ei_hbm.at[pl.ds(E + off, CHUNK)], didx)
      pltpu.async_copy(t_hbm.at[sidx], g, sem).wait()
      pltpu.async_copy(adt_hbm.at[didx], gd, sem).wait()

      def edge_body(e, carry2):
        va = g[e, pl.ds(cw, 16)]
        vd = gd[e, :]
        z = va + vd
        z = jnp.maximum(z, 0.2 * z)
        p = jnp.exp(z)
        pbuf[e, :] = p
        e_vec = jnp.full((16,), e, jnp.int32)
        for g_ in range(ngrp):
          if dh < 16:
            pe = plsc.load_gather(pbuf, [e_vec, pidx[g_]])
          else:
            pe = p
          msg[e, pl.ds(16 * g_, 16)] = g[e, pl.ds(16 * g_, 16)] * pe
        return carry2

      lax.fori_loop(0, CHUNK, edge_body, 0)
      pltpu.sync_copy(msg, acc_h.at[didx], add=True)
      pltpu.sync_copy(pbuf, acc_p.at[didx], add=True)
      return carry

    lax.fori_loop(0, nch, chunk_body, 0)
    plsc.subcore_barrier()
    pltpu.sync_copy(acc_h.at[pl.ds(r0, RPT)], ph_hbm.at[c, pl.ds(r0, RPT)])
    pltpu.sync_copy(acc_p.at[pl.ds(r0, RPT)], pp_hbm.at[c, pl.ds(r0, RPT)])

    @pl.when(s == NS - 1)
    def _():
      pltpu.sync_copy(acc_h.at[pl.ds(TAIL0, TAILR)],
                      ph_hbm.at[c, pl.ds(TAIL0, TAILR)])
      pltpu.sync_copy(acc_p.at[pl.ds(TAIL0, TAILR)],
                      pp_hbm.at[c, pl.ds(TAIL0, TAILR)])

  return k


_edge1 = _edge_kernel(C1, DIM_H)
_edge2 = _edge_kernel(DIM_OUT, DIM_OUT)


def _tc_pre(x_ref, w1_ref, a0s_ref, a0d_ref, t1_ref, adt_ref):
  h = jnp.dot(x_ref[...], w1_ref[...], preferred_element_type=jnp.float32)
  asrc = jnp.dot(h, a0s_ref[...], preferred_element_type=jnp.float32)
  adst = jnp.dot(h, a0d_ref[...], preferred_element_type=jnp.float32)
  t1_ref[...] = jnp.concatenate([h, asrc, asrc], axis=1)
  adt_ref[...] = jnp.concatenate([adst, adst], axis=1)


def _tc_mid(ph_ref, pp_ref, w2_ref, a2s_ref, a2d_ref, t2_ref, ad2_ref):
  num = ph_ref[0] + ph_ref[1]
  den = pp_ref[0] + pp_ref[1]
  s8 = den[:, :HEADS]
  rep = (lax.broadcasted_iota(jnp.int32, (HEADS, C1), 0)
         == lax.broadcasted_iota(jnp.int32, (HEADS, C1), 1) // DIM_H
         ).astype(jnp.float32)
  srep = jnp.dot(s8, rep, preferred_element_type=jnp.float32)
  out1 = num / (srep + 1e-16)
  gq = jnp.where(out1 > 0, out1, jnp.exp(jnp.minimum(out1, 0.0)) - 1.0)
  h2 = jnp.dot(gq, w2_ref[...], preferred_element_type=jnp.float32)
  as2 = jnp.sum(h2 * a2s_ref[...], axis=1, keepdims=True)
  ad2 = jnp.sum(h2 * a2d_ref[...], axis=1, keepdims=True)
  zeros = jnp.zeros((N, 16), jnp.float32)
  t2_ref[...] = jnp.concatenate([h2, as2 + zeros], axis=1)
  ad2_ref[...] = ad2 + zeros


def _tc_post(ph_ref, pp_ref, out_ref, ls_ref):
  num = ph_ref[0] + ph_ref[1]
  den = pp_ref[0] + pp_ref[1]
  out2 = num / (den + 1e-16)
  m = jnp.max(out2, axis=1, keepdims=True)
  lse = jnp.log(jnp.sum(jnp.exp(out2 - m), axis=1, keepdims=True)) + m
  out_ref[...] = out2
  ls_ref[...] = out2 - lse


@jax.jit
def kernel(x, edge_index, W1, a_src1, a_dst1, W2, a_src2, a_dst2):
  eye = jnp.eye(DIM_H, dtype=jnp.float32)
  a0s = (a_src1[:, :, None] * eye[:, None, :]).reshape(C1, HEADS)
  a0d = (a_dst1[:, :, None] * eye[:, None, :]).reshape(C1, HEADS)

  t1, adt = pl.pallas_call(
      _tc_pre,
      out_shape=(
          jax.ShapeDtypeStruct((N, C1 + 16), jnp.float32),
          jax.ShapeDtypeStruct((N, 16), jnp.float32),
      ),
  )(x, W1, a0s, a0d)

  ei = edge_index.reshape(-1)
  zh1 = jnp.zeros((RPT, C1), jnp.float32)
  zp = jnp.zeros((RPT, 16), jnp.float32)
  ph1, pp1 = _edge1(t1, adt, ei, zh1, zp)

  t2, ad2 = pl.pallas_call(
      _tc_mid,
      out_shape=(
          jax.ShapeDtypeStruct((N, 32), jnp.float32),
          jax.ShapeDtypeStruct((N, 16), jnp.float32),
      ),
  )(ph1, pp1, W2, a_src2, a_dst2)

  zh2 = jnp.zeros((RPT, DIM_OUT), jnp.float32)
  ph2, pp2 = _edge2(t2, ad2, ei, zh2, zp)

  out2, ls = pl.pallas_call(
      _tc_post,
      out_shape=(
          jax.ShapeDtypeStruct((N, DIM_OUT), jnp.float32),
          jax.ShapeDtypeStruct((N, DIM_OUT), jnp.float32),
      ),
  )(ph2, pp2)
  return (out2, ls)


# speedup vs baseline: 68.8746x; 1.5310x over previous
import functools

import jax
import jax.numpy as jnp
from jax import lax
from jax.experimental import pallas as pl
from jax.experimental.pallas import tpu as pltpu
from jax.experimental.pallas import tpu_sc as plsc

N = 10000
E = 320000
DIM_IN = 128
DIM_H = 8
HEADS = 8
C1 = HEADS * DIM_H
DIM_OUT = 16

NC = 2
NS = 16
NW = NC * NS
CHUNK = 128
NROW = E // CHUNK
BASE_ST = NROW // NW
REM_ST = NROW % NW
MAXST = BASE_ST + 1
NPAIRS = (MAXST + 1) // 2
RPT = 624
TAIL0 = NS * RPT
TAILR = N - TAIL0


def _edge_kernel(cw, dh):
  tw = cw + 16
  ngrp = cw // 16
  hpg = 16 // dh

  mesh = plsc.VectorSubcoreMesh(core_axis_name="c", subcore_axis_name="s")

  @functools.partial(
      pl.kernel,
      out_type=(
          jax.ShapeDtypeStruct((NC, N, cw), jnp.float32),
          jax.ShapeDtypeStruct((NC, N, 16), jnp.float32),
      ),
      mesh=mesh,
      scratch_types=[
          pltpu.VMEM((MAXST, CHUNK), jnp.int32),
          pltpu.VMEM((MAXST, CHUNK), jnp.int32),
          pltpu.VMEM((2, CHUNK, tw), jnp.float32),
          pltpu.VMEM((2, CHUNK, 16), jnp.float32),
          pltpu.VMEM((2, CHUNK, cw), jnp.float32),
          pltpu.VMEM((2, CHUNK, 16), jnp.float32),
          pltpu.VMEM_SHARED((N, cw), jnp.float32),
          pltpu.VMEM_SHARED((N, 16), jnp.float32),
          pltpu.SemaphoreType.DMA,
          pltpu.SemaphoreType.DMA,
          pltpu.SemaphoreType.DMA,
          pltpu.SemaphoreType.DMA,
      ],
      compiler_params=pltpu.CompilerParams(
          use_tc_tiling_on_sc=False, needs_layout_passes=False
      ),
  )
  def k(t_hbm, adt_hbm, ei_hbm, zh_hbm, zp_hbm, ph_hbm, pp_hbm,
        sidx, didx, g, gd, msg, pbuf, acc_h, acc_p,
        gsem0, gsem1, ssem0, ssem1):
    gsem = (gsem0, gsem1)
    ssem = (ssem0, ssem1)
    c = lax.axis_index("c")
    s = lax.axis_index("s")
    w = s * NC + c
    r0 = s * RPT

    pltpu.sync_copy(zh_hbm, acc_h.at[pl.ds(r0, RPT)])
    pltpu.sync_copy(zp_hbm, acc_p.at[pl.ds(r0, RPT)])

    @pl.when(s == NS - 1)
    def _():
      pltpu.sync_copy(zh_hbm.at[pl.ds(0, TAILR)], acc_h.at[pl.ds(TAIL0, TAILR)])
      pltpu.sync_copy(zp_hbm.at[pl.ds(0, TAILR)], acc_p.at[pl.ds(TAIL0, TAILR)])

    plsc.subcore_barrier()

    lane = lax.iota(jnp.int32, 16)
    shift = dh.bit_length() - 1
    pidx = [(lane >> shift) + g_ * hpg for g_ in range(ngrp)]

    rw0 = w * BASE_ST + jnp.minimum(w, REM_ST)
    pltpu.sync_copy(ei_hbm.at[0, pl.ds(rw0, BASE_ST)],
                    sidx.at[pl.ds(0, BASE_ST)])
    pltpu.sync_copy(ei_hbm.at[1, pl.ds(rw0, BASE_ST)],
                    didx.at[pl.ds(0, BASE_ST)])

    @pl.when(w < REM_ST)
    def _():
      pltpu.sync_copy(ei_hbm.at[0, rw0 + BASE_ST], sidx.at[BASE_ST])
      pltpu.sync_copy(ei_hbm.at[1, rw0 + BASE_ST], didx.at[BASE_ST])

    def issue_gathers(b, i):
      pltpu.async_copy(t_hbm.at[sidx.at[i]], g.at[b], gsem[b])
      pltpu.async_copy(adt_hbm.at[didx.at[i]], gd.at[b], gsem[b])

    def wait_gathers(b, i):
      pltpu.make_async_copy(t_hbm.at[sidx.at[i]], g.at[b], gsem[b]).wait()
      pltpu.make_async_copy(adt_hbm.at[didx.at[i]], gd.at[b], gsem[b]).wait()

    def issue_scatters(b, i):
      pltpu.async_copy(msg.at[b], acc_h.at[didx.at[i]], ssem[b], add=True)
      pltpu.async_copy(pbuf.at[b], acc_p.at[didx.at[i]], ssem[b], add=True)

    def wait_scatters(b, i):
      pltpu.make_async_copy(msg.at[b], acc_h.at[didx.at[i]], ssem[b]).wait()
      pltpu.make_async_copy(pbuf.at[b], acc_p.at[didx.at[i]], ssem[b]).wait()

    def compute(b):
      b_vec = jnp.full((16,), b, jnp.int32)

      def edge_body(e, carry2):
        va = g[b, e, pl.ds(cw, 16)]
        vd = gd[b, e, :]
        z = va + vd
        z = jnp.maximum(z, 0.2 * z)
        p = jnp.exp(z)
        pbuf[b, e, :] = p
        e_vec = jnp.full((16,), e, jnp.int32)
        for g_ in range(ngrp):
          if dh < 16:
            pe = plsc.load_gather(pbuf, [b_vec, e_vec, pidx[g_]])
          else:
            pe = p
          msg[b, e, pl.ds(16 * g_, 16)] = g[b, e, pl.ds(16 * g_, 16)] * pe
        return carry2

      lax.fori_loop(0, CHUNK, edge_body, 0, unroll=2)

    ns = BASE_ST + jnp.where(w < REM_ST, 1, 0)

    def handle(b, i):
      nb = 1 - b

      @pl.when(i + 1 < ns)
      def _():
        @pl.when(i >= 1)
        def _():
          wait_scatters(nb, i - 1)

        issue_gathers(nb, i + 1)

      @pl.when(i < ns)
      def _():
        wait_gathers(b, i)
        compute(b)
        issue_scatters(b, i)

    issue_gathers(0, 0)

    def pair_body(kk, carry):
      handle(0, 2 * kk)
      handle(1, 2 * kk + 1)
      return carry

    lax.fori_loop(0, NPAIRS, pair_body, 0)
    wait_scatters(1, 0)
    wait_scatters(0, 0)
    plsc.subcore_barrier()
    pltpu.sync_copy(acc_h.at[pl.ds(r0, RPT)], ph_hbm.at[c, pl.ds(r0, RPT)])
    pltpu.sync_copy(acc_p.at[pl.ds(r0, RPT)], pp_hbm.at[c, pl.ds(r0, RPT)])

    @pl.when(s == NS - 1)
    def _():
      pltpu.sync_copy(acc_h.at[pl.ds(TAIL0, TAILR)],
                      ph_hbm.at[c, pl.ds(TAIL0, TAILR)])
      pltpu.sync_copy(acc_p.at[pl.ds(TAIL0, TAILR)],
                      pp_hbm.at[c, pl.ds(TAIL0, TAILR)])

  return k


_edge1 = _edge_kernel(C1, DIM_H)
_edge2 = _edge_kernel(DIM_OUT, DIM_OUT)


def _tc_pre(x_ref, w1_ref, a0s_ref, a0d_ref, t1_ref, adt_ref):
  h = jnp.dot(x_ref[...], w1_ref[...], preferred_element_type=jnp.float32)
  asrc = jnp.dot(h, a0s_ref[...], preferred_element_type=jnp.float32)
  adst = jnp.dot(h, a0d_ref[...], preferred_element_type=jnp.float32)
  t1_ref[...] = jnp.concatenate([h, asrc, asrc], axis=1)
  adt_ref[...] = jnp.concatenate([adst, adst], axis=1)


def _tc_mid(ph_ref, pp_ref, w2_ref, a2s_ref, a2d_ref, t2_ref, ad2_ref):
  num = ph_ref[0] + ph_ref[1]
  den = pp_ref[0] + pp_ref[1]
  s8 = den[:, :HEADS]
  rep = (lax.broadcasted_iota(jnp.int32, (HEADS, C1), 0)
         == lax.broadcasted_iota(jnp.int32, (HEADS, C1), 1) // DIM_H
         ).astype(jnp.float32)
  srep = jnp.dot(s8, rep, preferred_element_type=jnp.float32)
  out1 = num / (srep + 1e-16)
  gq = jnp.where(out1 > 0, out1, jnp.exp(jnp.minimum(out1, 0.0)) - 1.0)
  h2 = jnp.dot(gq, w2_ref[...], preferred_element_type=jnp.float32)
  as2 = jnp.sum(h2 * a2s_ref[...], axis=1, keepdims=True)
  ad2 = jnp.sum(h2 * a2d_ref[...], axis=1, keepdims=True)
  zeros = jnp.zeros((N, 16), jnp.float32)
  t2_ref[...] = jnp.concatenate([h2, as2 + zeros], axis=1)
  ad2_ref[...] = ad2 + zeros


def _tc_post(ph_ref, pp_ref, out_ref, ls_ref):
  num = ph_ref[0] + ph_ref[1]
  den = pp_ref[0] + pp_ref[1]
  out2 = num / (den + 1e-16)
  m = jnp.max(out2, axis=1, keepdims=True)
  lse = jnp.log(jnp.sum(jnp.exp(out2 - m), axis=1, keepdims=True)) + m
  out_ref[...] = out2
  ls_ref[...] = out2 - lse


@jax.jit
def kernel(x, edge_index, W1, a_src1, a_dst1, W2, a_src2, a_dst2):
  eye = jnp.eye(DIM_H, dtype=jnp.float32)
  a0s = (a_src1[:, :, None] * eye[:, None, :]).reshape(C1, HEADS)
  a0d = (a_dst1[:, :, None] * eye[:, None, :]).reshape(C1, HEADS)

  t1, adt = pl.pallas_call(
      _tc_pre,
      out_shape=(
          jax.ShapeDtypeStruct((N, C1 + 16), jnp.float32),
          jax.ShapeDtypeStruct((N, 16), jnp.float32),
      ),
  )(x, W1, a0s, a0d)

  ei = edge_index.reshape(2, NROW, CHUNK)
  zh1 = jnp.zeros((RPT, C1), jnp.float32)
  zp = jnp.zeros((RPT, 16), jnp.float32)
  ph1, pp1 = _edge1(t1, adt, ei, zh1, zp)

  t2, ad2 = pl.pallas_call(
      _tc_mid,
      out_shape=(
          jax.ShapeDtypeStruct((N, 32), jnp.float32),
          jax.ShapeDtypeStruct((N, 16), jnp.float32),
      ),
  )(ph1, pp1, W2, a_src2, a_dst2)

  zh2 = jnp.zeros((RPT, DIM_OUT), jnp.float32)
  ph2, pp2 = _edge2(t2, ad2, ei, zh2, zp)

  out2, ls = pl.pallas_call(
      _tc_post,
      out_shape=(
          jax.ShapeDtypeStruct((N, DIM_OUT), jnp.float32),
          jax.ShapeDtypeStruct((N, DIM_OUT), jnp.float32),
      ),
  )(ph2, pp2)
  return (out2, ls)


# speedup vs baseline: 170.3718x; 2.4737x over previous
import functools

import jax
import jax.numpy as jnp
from jax import lax
from jax.experimental import pallas as pl
from jax.experimental.pallas import tpu as pltpu
from jax.experimental.pallas import tpu_sc as plsc

N = 10000
E = 320000
DIM_IN = 128
DIM_H = 8
HEADS = 8
C1 = HEADS * DIM_H
DIM_OUT = 16

NC = 2
NS = 16
NW = NC * NS
CHUNK = 128
NROW = E // CHUNK
BASE_ST = NROW // NW
REM_ST = NROW % NW
MAXST = BASE_ST + 1
NPAIRS = (MAXST + 1) // 2
RPT = 624
TAIL0 = NS * RPT
TAILR = N - TAIL0


def _edge_kernel(cw, dh):
  tw = cw + 16
  ngrp = cw // 16
  hpg = 16 // dh

  mesh = plsc.VectorSubcoreMesh(core_axis_name="c", subcore_axis_name="s")

  @functools.partial(
      pl.kernel,
      out_type=jax.ShapeDtypeStruct((NC, N, tw), jnp.float32),
      mesh=mesh,
      scratch_types=[
          pltpu.VMEM((MAXST, CHUNK), jnp.int32),
          pltpu.VMEM((MAXST, CHUNK), jnp.int32),
          pltpu.VMEM((2, CHUNK, tw), jnp.float32),
          pltpu.VMEM((2, CHUNK, 16), jnp.float32),
          pltpu.VMEM((2, CHUNK, tw), jnp.float32),
          pltpu.VMEM_SHARED((N, tw), jnp.float32),
          pltpu.SemaphoreType.DMA,
          pltpu.SemaphoreType.DMA,
          pltpu.SemaphoreType.DMA,
          pltpu.SemaphoreType.DMA,
      ],
      compiler_params=pltpu.CompilerParams(
          use_tc_tiling_on_sc=False, needs_layout_passes=False
      ),
  )
  def k(t_hbm, adt_hbm, ei_hbm, z_hbm, pacc_hbm,
        sidx, didx, g, gd, msg, acc,
        gsem0, gsem1, ssem0, ssem1):
    gsem = (gsem0, gsem1)
    ssem = (ssem0, ssem1)
    c = lax.axis_index("c")
    s = lax.axis_index("s")
    w = s * NC + c
    r0 = s * RPT

    pltpu.sync_copy(z_hbm, acc.at[pl.ds(r0, RPT)])

    @pl.when(s == NS - 1)
    def _():
      pltpu.sync_copy(z_hbm.at[pl.ds(0, TAILR)], acc.at[pl.ds(TAIL0, TAILR)])

    plsc.subcore_barrier()

    lane = lax.iota(jnp.int32, 16)
    shift = dh.bit_length() - 1
    pidx = [(lane >> shift) + g_ * hpg for g_ in range(ngrp)]

    rw0 = w * BASE_ST + jnp.minimum(w, REM_ST)
    pltpu.sync_copy(ei_hbm.at[0, pl.ds(rw0, BASE_ST)],
                    sidx.at[pl.ds(0, BASE_ST)])
    pltpu.sync_copy(ei_hbm.at[1, pl.ds(rw0, BASE_ST)],
                    didx.at[pl.ds(0, BASE_ST)])

    @pl.when(w < REM_ST)
    def _():
      pltpu.sync_copy(ei_hbm.at[0, rw0 + BASE_ST], sidx.at[BASE_ST])
      pltpu.sync_copy(ei_hbm.at[1, rw0 + BASE_ST], didx.at[BASE_ST])

    def issue_gathers(b, i):
      pltpu.async_copy(t_hbm.at[sidx.at[i]], g.at[b], gsem[b])
      pltpu.async_copy(adt_hbm.at[didx.at[i]], gd.at[b], gsem[b])

    def wait_gathers(b, i):
      pltpu.make_async_copy(t_hbm.at[sidx.at[i]], g.at[b], gsem[b]).wait()
      pltpu.make_async_copy(adt_hbm.at[didx.at[i]], gd.at[b], gsem[b]).wait()

    def issue_scatters(b, i):
      pltpu.async_copy(msg.at[b], acc.at[didx.at[i]], ssem[b], add=True)

    def wait_scatters(b, i):
      pltpu.make_async_copy(msg.at[b], acc.at[didx.at[i]], ssem[b]).wait()

    def compute(b):
      b_vec = jnp.full((16,), b, jnp.int32)

      @plsc.parallel_loop(0, CHUNK, unroll=2)
      def _(e):
        va = g[b, e, pl.ds(cw, 16)]
        vd = gd[b, e, :]
        z = va + vd
        z = jnp.maximum(z, 0.2 * z)
        msg[b, e, pl.ds(cw, 16)] = jnp.exp(z)

      @plsc.parallel_loop(0, CHUNK, unroll=2)
      def _(e):
        if dh < 16:
          e_vec = jnp.full((16,), e, jnp.int32)
          for g_ in range(ngrp):
            pe = plsc.load_gather(msg, [b_vec, e_vec, pidx[g_] + cw])
            msg[b, e, pl.ds(16 * g_, 16)] = g[b, e, pl.ds(16 * g_, 16)] * pe
        else:
          pe = msg[b, e, pl.ds(cw, 16)]
          msg[b, e, pl.ds(0, 16)] = g[b, e, pl.ds(0, 16)] * pe

    ns = BASE_ST + jnp.where(w < REM_ST, 1, 0)

    def handle(b, i):
      nb = 1 - b

      @pl.when(i + 1 < ns)
      def _():
        @pl.when(i >= 1)
        def _():
          wait_scatters(nb, i - 1)

        issue_gathers(nb, i + 1)

      @pl.when(i < ns)
      def _():
        wait_gathers(b, i)
        compute(b)
        issue_scatters(b, i)

    issue_gathers(0, 0)

    def pair_body(kk, carry):
      handle(0, 2 * kk)
      handle(1, 2 * kk + 1)
      return carry

    lax.fori_loop(0, NPAIRS, pair_body, 0)
    wait_scatters(1, 0)
    wait_scatters(0, 0)
    plsc.subcore_barrier()
    pltpu.sync_copy(acc.at[pl.ds(r0, RPT)], pacc_hbm.at[c, pl.ds(r0, RPT)])

    @pl.when(s == NS - 1)
    def _():
      pltpu.sync_copy(acc.at[pl.ds(TAIL0, TAILR)],
                      pacc_hbm.at[c, pl.ds(TAIL0, TAILR)])

  return k


_edge1 = _edge_kernel(C1, DIM_H)
_edge2 = _edge_kernel(DIM_OUT, DIM_OUT)


def _tc_pre(x_ref, w1_ref, a0s_ref, a0d_ref, t1_ref, adt_ref):
  h = jnp.dot(x_ref[...], w1_ref[...], preferred_element_type=jnp.float32)
  asrc = jnp.dot(h, a0s_ref[...], preferred_element_type=jnp.float32)
  adst = jnp.dot(h, a0d_ref[...], preferred_element_type=jnp.float32)
  t1_ref[...] = jnp.concatenate([h, asrc, asrc], axis=1)
  adt_ref[...] = jnp.concatenate([adst, adst], axis=1)


def _tc_mid(pacc_ref, w2_ref, a2s_ref, a2d_ref, t2_ref, ad2_ref):
  both = pacc_ref[0] + pacc_ref[1]
  num = both[:, :C1]
  s8 = both[:, C1:C1 + HEADS]
  rep = (lax.broadcasted_iota(jnp.int32, (HEADS, C1), 0)
         == lax.broadcasted_iota(jnp.int32, (HEADS, C1), 1) // DIM_H
         ).astype(jnp.float32)
  srep = jnp.dot(s8, rep, preferred_element_type=jnp.float32)
  out1 = num / (srep + 1e-16)
  gq = jnp.where(out1 > 0, out1, jnp.exp(jnp.minimum(out1, 0.0)) - 1.0)
  h2 = jnp.dot(gq, w2_ref[...], preferred_element_type=jnp.float32)
  as2 = jnp.sum(h2 * a2s_ref[...], axis=1, keepdims=True)
  ad2 = jnp.sum(h2 * a2d_ref[...], axis=1, keepdims=True)
  zeros = jnp.zeros((N, 16), jnp.float32)
  t2_ref[...] = jnp.concatenate([h2, as2 + zeros], axis=1)
  ad2_ref[...] = ad2 + zeros


def _tc_post(pacc_ref, out_ref, ls_ref):
  both = pacc_ref[0] + pacc_ref[1]
  out2 = both[:, :DIM_OUT] / (both[:, DIM_OUT:] + 1e-16)
  m = jnp.max(out2, axis=1, keepdims=True)
  lse = jnp.log(jnp.sum(jnp.exp(out2 - m), axis=1, keepdims=True)) + m
  out_ref[...] = out2
  ls_ref[...] = out2 - lse


@jax.jit
def kernel(x, edge_index, W1, a_src1, a_dst1, W2, a_src2, a_dst2):
  eye = jnp.eye(DIM_H, dtype=jnp.float32)
  a0s = (a_src1[:, :, None] * eye[:, None, :]).reshape(C1, HEADS)
  a0d = (a_dst1[:, :, None] * eye[:, None, :]).reshape(C1, HEADS)

  t1, adt = pl.pallas_call(
      _tc_pre,
      out_shape=(
          jax.ShapeDtypeStruct((N, C1 + 16), jnp.float32),
          jax.ShapeDtypeStruct((N, 16), jnp.float32),
      ),
  )(x, W1, a0s, a0d)

  ei = edge_index.reshape(2, NROW, CHUNK)
  z1 = jnp.zeros((RPT, C1 + 16), jnp.float32)
  pacc1 = _edge1(t1, adt, ei, z1)

  t2, ad2 = pl.pallas_call(
      _tc_mid,
      out_shape=(
          jax.ShapeDtypeStruct((N, 32), jnp.float32),
          jax.ShapeDtypeStruct((N, 16), jnp.float32),
      ),
  )(pacc1, W2, a_src2, a_dst2)

  z2 = jnp.zeros((RPT, DIM_OUT + 16), jnp.float32)
  pacc2 = _edge2(t2, ad2, ei, z2)

  out2, ls = pl.pallas_call(
      _tc_post,
      out_shape=(
          jax.ShapeDtypeStruct((N, DIM_OUT), jnp.float32),
          jax.ShapeDtypeStruct((N, DIM_OUT), jnp.float32),
      ),
  )(pacc2)
  return (out2, ls)


# speedup vs baseline: 191.4333x; 1.1236x over previous
import functools

import jax
import jax.numpy as jnp
from jax import lax
from jax.experimental import pallas as pl
from jax.experimental.pallas import tpu as pltpu
from jax.experimental.pallas import tpu_sc as plsc

N = 10000
E = 320000
DIM_IN = 128
DIM_H = 8
HEADS = 8
C1 = HEADS * DIM_H
DIM_OUT = 16

NC = 2
NS = 16
NW = NC * NS
CHUNK = 128
NROW = E // CHUNK
BASE_ST = NROW // NW
REM_ST = NROW % NW
MAXST = BASE_ST + 1
NPAIRS = (MAXST + 1) // 2
RPT = 624
TAIL0 = NS * RPT
TAILR = N - TAIL0


def _edge_kernel(cw, dh):
  tw = cw + 16
  ngrp = cw // 16
  hpg = 16 // dh

  mesh = plsc.VectorSubcoreMesh(core_axis_name="c", subcore_axis_name="s")

  @functools.partial(
      pl.kernel,
      out_type=jax.ShapeDtypeStruct((NC, N, tw), jnp.float32),
      mesh=mesh,
      scratch_types=[
          pltpu.VMEM((MAXST, CHUNK), jnp.int32),
          pltpu.VMEM((MAXST, CHUNK), jnp.int32),
          pltpu.VMEM((2, CHUNK, tw), jnp.float32),
          pltpu.VMEM((2, CHUNK, 16), jnp.float32),
          pltpu.VMEM((2, CHUNK, tw), jnp.float32),
          pltpu.VMEM_SHARED((N, tw), jnp.float32),
          pltpu.SemaphoreType.DMA,
          pltpu.SemaphoreType.DMA,
          pltpu.SemaphoreType.DMA,
          pltpu.SemaphoreType.DMA,
      ],
      compiler_params=pltpu.CompilerParams(
          use_tc_tiling_on_sc=False, needs_layout_passes=False
      ),
  )
  def k(t_hbm, adt_hbm, ei_hbm, z_hbm, pacc_hbm,
        sidx, didx, g, gd, msg, acc,
        gsem0, gsem1, ssem0, ssem1):
    gsem = (gsem0, gsem1)
    ssem = (ssem0, ssem1)
    c = lax.axis_index("c")
    s = lax.axis_index("s")
    w = s * NC + c
    r0 = s * RPT

    pltpu.sync_copy(z_hbm, acc.at[pl.ds(r0, RPT)])

    @pl.when(s == NS - 1)
    def _():
      pltpu.sync_copy(z_hbm.at[pl.ds(0, TAILR)], acc.at[pl.ds(TAIL0, TAILR)])

    plsc.subcore_barrier()

    lane = lax.iota(jnp.int32, 16)
    shift = dh.bit_length() - 1
    pidx = [(lane >> shift) + g_ * hpg for g_ in range(ngrp)]

    rw0 = w * BASE_ST + jnp.minimum(w, REM_ST)
    pltpu.sync_copy(ei_hbm.at[0, pl.ds(rw0, BASE_ST)],
                    sidx.at[pl.ds(0, BASE_ST)])
    pltpu.sync_copy(ei_hbm.at[1, pl.ds(rw0, BASE_ST)],
                    didx.at[pl.ds(0, BASE_ST)])

    @pl.when(w < REM_ST)
    def _():
      pltpu.sync_copy(ei_hbm.at[0, rw0 + BASE_ST], sidx.at[BASE_ST])
      pltpu.sync_copy(ei_hbm.at[1, rw0 + BASE_ST], didx.at[BASE_ST])

    def issue_gathers(b, i):
      pltpu.async_copy(t_hbm.at[sidx.at[i]], g.at[b], gsem[b])
      pltpu.async_copy(adt_hbm.at[didx.at[i]], gd.at[b], gsem[b])

    def wait_gathers(b, i):
      pltpu.make_async_copy(t_hbm.at[sidx.at[i]], g.at[b], gsem[b]).wait()
      pltpu.make_async_copy(adt_hbm.at[didx.at[i]], gd.at[b], gsem[b]).wait()

    def issue_scatters(b, i):
      pltpu.async_copy(msg.at[b], acc.at[didx.at[i]], ssem[b], add=True)

    def wait_scatters(b, i):
      pltpu.make_async_copy(msg.at[b], acc.at[didx.at[i]], ssem[b]).wait()

    def compute(b):
      b_vec = jnp.full((16,), b, jnp.int32)

      @plsc.parallel_loop(0, CHUNK, unroll=4)
      def _(e):
        va = g[b, e, pl.ds(cw, 16)]
        vd = gd[b, e, :]
        z = va + vd
        z = jnp.maximum(z, 0.2 * z)
        msg[b, e, pl.ds(cw, 16)] = jnp.exp(z)

      @plsc.parallel_loop(0, CHUNK, unroll=4)
      def _(e):
        if dh < 16:
          e_vec = jnp.full((16,), e, jnp.int32)
          for g_ in range(ngrp):
            pe = plsc.load_gather(msg, [b_vec, e_vec, pidx[g_] + cw])
            msg[b, e, pl.ds(16 * g_, 16)] = g[b, e, pl.ds(16 * g_, 16)] * pe
        else:
          pe = msg[b, e, pl.ds(cw, 16)]
          msg[b, e, pl.ds(0, 16)] = g[b, e, pl.ds(0, 16)] * pe

    ns = BASE_ST + jnp.where(w < REM_ST, 1, 0)

    def handle(b, i):
      nb = 1 - b

      @pl.when(i + 1 < ns)
      def _():
        issue_gathers(nb, i + 1)

      @pl.when(i < ns)
      def _():
        wait_gathers(b, i)

        @pl.when(i >= 2)
        def _():
          wait_scatters(b, i - 2)

        compute(b)
        issue_scatters(b, i)

    issue_gathers(0, 0)

    def pair_body(kk, carry):
      handle(0, 2 * kk)
      handle(1, 2 * kk + 1)
      return carry

    lax.fori_loop(0, NPAIRS, pair_body, 0)
    wait_scatters(1, 0)
    wait_scatters(0, 0)
    plsc.subcore_barrier()
    pltpu.sync_copy(acc.at[pl.ds(r0, RPT)], pacc_hbm.at[c, pl.ds(r0, RPT)])

    @pl.when(s == NS - 1)
    def _():
      pltpu.sync_copy(acc.at[pl.ds(TAIL0, TAILR)],
                      pacc_hbm.at[c, pl.ds(TAIL0, TAILR)])

  return k


_edge1 = _edge_kernel(C1, DIM_H)
_edge2 = _edge_kernel(DIM_OUT, DIM_OUT)


def _tc_pre(x_ref, w1_ref, a0s_ref, a0d_ref, t1_ref, adt_ref):
  h = jnp.dot(x_ref[...], w1_ref[...], preferred_element_type=jnp.float32)
  asrc = jnp.dot(h, a0s_ref[...], preferred_element_type=jnp.float32)
  adst = jnp.dot(h, a0d_ref[...], preferred_element_type=jnp.float32)
  t1_ref[...] = jnp.concatenate([h, asrc, asrc], axis=1)
  adt_ref[...] = jnp.concatenate([adst, adst], axis=1)


def _tc_mid(pacc_ref, w2_ref, a2s_ref, a2d_ref, t2_ref, ad2_ref):
  both = pacc_ref[0] + pacc_ref[1]
  num = both[:, :C1]
  s8 = both[:, C1:C1 + HEADS]
  rep = (lax.broadcasted_iota(jnp.int32, (HEADS, C1), 0)
         == lax.broadcasted_iota(jnp.int32, (HEADS, C1), 1) // DIM_H
         ).astype(jnp.float32)
  srep = jnp.dot(s8, rep, preferred_element_type=jnp.float32)
  out1 = num / (srep + 1e-16)
  gq = jnp.where(out1 > 0, out1, jnp.exp(jnp.minimum(out1, 0.0)) - 1.0)
  h2 = jnp.dot(gq, w2_ref[...], preferred_element_type=jnp.float32)
  as2 = jnp.sum(h2 * a2s_ref[...], axis=1, keepdims=True)
  ad2 = jnp.sum(h2 * a2d_ref[...], axis=1, keepdims=True)
  zeros = jnp.zeros((N, 16), jnp.float32)
  t2_ref[...] = jnp.concatenate([h2, as2 + zeros], axis=1)
  ad2_ref[...] = ad2 + zeros


def _tc_post(pacc_ref, out_ref, ls_ref):
  both = pacc_ref[0] + pacc_ref[1]
  out2 = both[:, :DIM_OUT] / (both[:, DIM_OUT:] + 1e-16)
  m = jnp.max(out2, axis=1, keepdims=True)
  lse = jnp.log(jnp.sum(jnp.exp(out2 - m), axis=1, keepdims=True)) + m
  out_ref[...] = out2
  ls_ref[...] = out2 - lse


@jax.jit
def kernel(x, edge_index, W1, a_src1, a_dst1, W2, a_src2, a_dst2):
  eye = jnp.eye(DIM_H, dtype=jnp.float32)
  a0s = (a_src1[:, :, None] * eye[:, None, :]).reshape(C1, HEADS)
  a0d = (a_dst1[:, :, None] * eye[:, None, :]).reshape(C1, HEADS)

  t1, adt = pl.pallas_call(
      _tc_pre,
      out_shape=(
          jax.ShapeDtypeStruct((N, C1 + 16), jnp.float32),
          jax.ShapeDtypeStruct((N, 16), jnp.float32),
      ),
  )(x, W1, a0s, a0d)

  ei = edge_index.reshape(2, NROW, CHUNK)
  z1 = jnp.zeros((RPT, C1 + 16), jnp.float32)
  pacc1 = _edge1(t1, adt, ei, z1)

  t2, ad2 = pl.pallas_call(
      _tc_mid,
      out_shape=(
          jax.ShapeDtypeStruct((N, 32), jnp.float32),
          jax.ShapeDtypeStruct((N, 16), jnp.float32),
      ),
  )(pacc1, W2, a_src2, a_dst2)

  z2 = jnp.zeros((RPT, DIM_OUT + 16), jnp.float32)
  pacc2 = _edge2(t2, ad2, ei, z2)

  out2, ls = pl.pallas_call(
      _tc_post,
      out_shape=(
          jax.ShapeDtypeStruct((N, DIM_OUT), jnp.float32),
          jax.ShapeDtypeStruct((N, DIM_OUT), jnp.float32),
      ),
  )(pacc2)
  return (out2, ls)
